# baseline (speedup 1.0000x reference)
import dataclasses
import functools

import jax
import jax.numpy as jnp
from jax.experimental import pallas as pl
from jax.experimental.pallas import tpu as pltpu
from jax.experimental.pallas import tpu_sc as plsc

_WINDOW = 128
_WORKERS = 32
_BE = 3200
_K = 5


def _sc_compiler_params():
    cp = pltpu.CompilerParams(use_tc_tiling_on_sc=False)
    if "needs_layout_passes" in pltpu.CompilerParams.__dataclass_fields__:
        cp = dataclasses.replace(cp, needs_layout_passes=False)
    return cp


def _sc_gather(table, eidx, npad, dim):
    mesh = plsc.VectorSubcoreMesh(core_axis_name="core", subcore_axis_name="subcore")
    half = _WINDOW // 2

    @functools.partial(
        pl.kernel,
        out_type=jax.ShapeDtypeStruct((npad, dim), table.dtype),
        mesh=mesh,
        scratch_types=[pltpu.VMEM((_WINDOW,), jnp.int32)],
        compiler_params=_sc_compiler_params(),
    )
    def gather_kernel(x_hbm, i_hbm, o_hbm, v_ref):
        def body(i_vmem, o_vmem):
            for c in range(half // 16):
                pos = jax.lax.iota(jnp.int32, 16) * 2 + c * 32
                plsc.store_scatter(v_ref, [pos], i_vmem.at[0][pl.ds(c * 16, 16)])
                plsc.store_scatter(v_ref, [pos + 1], i_vmem.at[1][pl.ds(c * 16, 16)])
            pltpu.sync_copy(x_hbm.at[v_ref], o_vmem)

        pltpu.emit_pipeline(
            body,
            grid=(npad // _WINDOW,),
            in_specs=[pl.BlockSpec((2, half), index_map=lambda i: (0, i))],
            out_specs=[pl.BlockSpec((_WINDOW, dim), index_map=lambda i: (i, 0))],
            core_axis_name=("core", "subcore"),
            dimension_semantics=(pltpu.PARALLEL,),
        )(i_hbm, o_hbm)

    return gather_kernel(table, eidx)


def _mlp_math(g, eft, dt, w1ab, w1c, w1d, w2, b1, b2, gamma, beta):
    x1 = (
        jax.lax.dot_general(w1ab, g, (((1,), (1,)), ((), ())),
                            preferred_element_type=jnp.float32)
        + jnp.dot(w1c, eft, preferred_element_type=jnp.float32)
        + jnp.dot(w1d, dt, preferred_element_type=jnp.float32)
        + b1
    )
    h = x1 * jax.nn.sigmoid(x1)
    x2 = jnp.dot(w2, h, preferred_element_type=jnp.float32) + b2
    x = eft + x2 * jax.nn.sigmoid(x2)
    mu = jnp.mean(x, axis=0, keepdims=True)
    xc = x - mu
    var = jnp.mean(xc * xc, axis=0, keepdims=True)
    y = xc * jax.lax.rsqrt(var + 1e-5)
    return y * gamma + beta


def _mlp_body(g_ref, eft_ref, dt_ref, w1ab_ref, w1c_ref, w1d_ref, w2_ref,
              b1_ref, b2_ref, gamma_ref, beta_ref, out_ref):
    out_ref[...] = _mlp_math(
        g_ref[...], eft_ref[...], dt_ref[...], w1ab_ref[...], w1c_ref[...],
        w1d_ref[...], w2_ref[...], b1_ref[...], b2_ref[...],
        gamma_ref[...], beta_ref[...])


def _mlp_body_carry(carry_ref, g_ref, eft_ref, dt_ref, w1ab_ref, w1c_ref,
                    w1d_ref, w2_ref, b1_ref, b2_ref, gamma_ref, beta_ref,
                    out_ref):
    del carry_ref
    out_ref[...] = _mlp_math(
        g_ref[...], eft_ref[...], dt_ref[...], w1ab_ref[...], w1c_ref[...],
        w1d_ref[...], w2_ref[...], b1_ref[...], b2_ref[...],
        gamma_ref[...], beta_ref[...])


def kernel(node_scalars, edge_feats, d, edge_index, W1, b1, W2, b2, gamma, beta):
    n_nodes, dim = node_scalars.shape
    e, f = edge_feats.shape
    r = d.shape[1]

    eft = edge_feats.T
    dt = d.T
    w1t = W1.T
    w1ab = w1t[:, 0 : 2 * dim]
    w1c = w1t[:, 2 * dim : 2 * dim + f]
    w1d = w1t[:, 2 * dim + f :]
    b1c = b1.reshape(f, 1)
    b2c = b2.reshape(f, 1)
    gac = gamma.reshape(f, 1)
    bec = beta.reshape(f, 1)

    ek = e // _K
    nbk = ek // _BE
    unit = _WINDOW * _WORKERS
    npadk = ((2 * ek + unit - 1) // unit) * unit

    gs = []
    for k in range(_K):
        eidx = edge_index[:, k * ek : (k + 1) * ek]
        eidx = jnp.pad(eidx, ((0, 0), (0, npadk // 2 - ek)))
        gs.append(_sc_gather(node_scalars, eidx, npadk, dim))

    out = None
    for k in range(_K):
        g128 = gs[k].reshape(npadk // 2, 2 * dim)
        base_specs = [
            pl.BlockSpec((_BE, 2 * dim), lambda i: (i, 0)),
            pl.BlockSpec((f, _BE), lambda i, k=k: (0, i + k * nbk)),
            pl.BlockSpec((r, _BE), lambda i, k=k: (0, i + k * nbk)),
            pl.BlockSpec((f, 2 * dim), lambda i: (0, 0)),
            pl.BlockSpec((f, f), lambda i: (0, 0)),
            pl.BlockSpec((f, r), lambda i: (0, 0)),
            pl.BlockSpec((f, f), lambda i: (0, 0)),
            pl.BlockSpec((f, 1), lambda i: (0, 0)),
            pl.BlockSpec((f, 1), lambda i: (0, 0)),
            pl.BlockSpec((f, 1), lambda i: (0, 0)),
            pl.BlockSpec((f, 1), lambda i: (0, 0)),
        ]
        out_spec = pl.BlockSpec((f, _BE), lambda i, k=k: (0, i + k * nbk))
        args = (g128, eft, dt, w1ab, w1c, w1d, W2.T, b1c, b2c, gac, bec)
        if k == 0:
            out = pl.pallas_call(
                _mlp_body,
                grid=(nbk,),
                in_specs=base_specs,
                out_specs=out_spec,
                out_shape=jax.ShapeDtypeStruct((f, e), jnp.float32),
                compiler_params=pltpu.CompilerParams(
                    dimension_semantics=("arbitrary",)),
            )(*args)
        else:
            carry_spec = pl.BlockSpec(
                (f, _BE), lambda i, k=k: (0, (k - 1) * nbk))
            out = pl.pallas_call(
                _mlp_body_carry,
                grid=(nbk,),
                in_specs=[carry_spec] + base_specs,
                out_specs=out_spec,
                out_shape=jax.ShapeDtypeStruct((f, e), jnp.float32),
                input_output_aliases={0: 0},
                compiler_params=pltpu.CompilerParams(
                    dimension_semantics=("arbitrary",)),
            )(out, *args)
    return out.T


# speedup vs baseline: 1.2256x; 1.2256x over previous
import dataclasses
import functools

import jax
import jax.numpy as jnp
from jax.experimental import pallas as pl
from jax.experimental.pallas import tpu as pltpu
from jax.experimental.pallas import tpu_sc as plsc

_WINDOW = 128
_WORKERS = 32
_BE = 3200
_K = 2


def _sc_compiler_params():
    cp = pltpu.CompilerParams(use_tc_tiling_on_sc=False)
    if "needs_layout_passes" in pltpu.CompilerParams.__dataclass_fields__:
        cp = dataclasses.replace(cp, needs_layout_passes=False)
    return cp


def _sc_gather(table, eidx, npad, dim):
    mesh = plsc.VectorSubcoreMesh(core_axis_name="core", subcore_axis_name="subcore")
    half = _WINDOW // 2

    @functools.partial(
        pl.kernel,
        out_type=jax.ShapeDtypeStruct((npad, dim), table.dtype),
        mesh=mesh,
        scratch_types=[pltpu.VMEM((_WINDOW,), jnp.int32)],
        compiler_params=_sc_compiler_params(),
    )
    def gather_kernel(x_hbm, i_hbm, o_hbm, v_ref):
        def body(i_vmem, o_vmem):
            for c in range(half // 16):
                pos = jax.lax.iota(jnp.int32, 16) * 2 + c * 32
                plsc.store_scatter(v_ref, [pos], i_vmem.at[0][pl.ds(c * 16, 16)])
                plsc.store_scatter(v_ref, [pos + 1], i_vmem.at[1][pl.ds(c * 16, 16)])
            pltpu.sync_copy(x_hbm.at[v_ref], o_vmem)

        pltpu.emit_pipeline(
            body,
            grid=(npad // _WINDOW,),
            in_specs=[pl.BlockSpec((2, half), index_map=lambda i: (0, i))],
            out_specs=[pl.BlockSpec((_WINDOW, dim), index_map=lambda i: (i, 0))],
            core_axis_name=("core", "subcore"),
            dimension_semantics=(pltpu.PARALLEL,),
        )(i_hbm, o_hbm)

    return gather_kernel(table, eidx)


def _mlp_math(g, eft, dt, w1ab, w1c, w1d, w2, b1, b2, gamma, beta):
    x1 = (
        jax.lax.dot_general(w1ab, g, (((1,), (1,)), ((), ())),
                            preferred_element_type=jnp.float32)
        + jnp.dot(w1c, eft, preferred_element_type=jnp.float32)
        + jnp.dot(w1d, dt, preferred_element_type=jnp.float32)
        + b1
    )
    h = x1 * jax.nn.sigmoid(x1)
    x2 = jnp.dot(w2, h, preferred_element_type=jnp.float32) + b2
    x = eft + x2 * jax.nn.sigmoid(x2)
    mu = jnp.mean(x, axis=0, keepdims=True)
    xc = x - mu
    var = jnp.mean(xc * xc, axis=0, keepdims=True)
    y = xc * jax.lax.rsqrt(var + 1e-5)
    return y * gamma + beta


def _mlp_body(g_ref, eft_ref, dt_ref, w1ab_ref, w1c_ref, w1d_ref, w2_ref,
              b1_ref, b2_ref, gamma_ref, beta_ref, out_ref):
    out_ref[...] = _mlp_math(
        g_ref[...], eft_ref[...], dt_ref[...], w1ab_ref[...], w1c_ref[...],
        w1d_ref[...], w2_ref[...], b1_ref[...], b2_ref[...],
        gamma_ref[...], beta_ref[...])


def _mlp_body_carry(carry_ref, g_ref, eft_ref, dt_ref, w1ab_ref, w1c_ref,
                    w1d_ref, w2_ref, b1_ref, b2_ref, gamma_ref, beta_ref,
                    out_ref):
    del carry_ref
    out_ref[...] = _mlp_math(
        g_ref[...], eft_ref[...], dt_ref[...], w1ab_ref[...], w1c_ref[...],
        w1d_ref[...], w2_ref[...], b1_ref[...], b2_ref[...],
        gamma_ref[...], beta_ref[...])


def kernel(node_scalars, edge_feats, d, edge_index, W1, b1, W2, b2, gamma, beta):
    n_nodes, dim = node_scalars.shape
    e, f = edge_feats.shape
    r = d.shape[1]

    eft = edge_feats.T
    dt = d.T
    w1t = W1.T
    w1ab = w1t[:, 0 : 2 * dim]
    w1c = w1t[:, 2 * dim : 2 * dim + f]
    w1d = w1t[:, 2 * dim + f :]
    b1c = b1.reshape(f, 1)
    b2c = b2.reshape(f, 1)
    gac = gamma.reshape(f, 1)
    bec = beta.reshape(f, 1)

    ek = e // _K
    nbk = ek // _BE
    unit = _WINDOW * _WORKERS
    npadk = ((2 * ek + unit - 1) // unit) * unit

    gs = []
    for k in range(_K):
        eidx = edge_index[:, k * ek : (k + 1) * ek]
        eidx = jnp.pad(eidx, ((0, 0), (0, npadk // 2 - ek)))
        gs.append(_sc_gather(node_scalars, eidx, npadk, dim))

    out = None
    for k in range(_K):
        g128 = gs[k].reshape(npadk // 2, 2 * dim)
        base_specs = [
            pl.BlockSpec((_BE, 2 * dim), lambda i: (i, 0)),
            pl.BlockSpec((f, _BE), lambda i, k=k: (0, i + k * nbk)),
            pl.BlockSpec((r, _BE), lambda i, k=k: (0, i + k * nbk)),
            pl.BlockSpec((f, 2 * dim), lambda i: (0, 0)),
            pl.BlockSpec((f, f), lambda i: (0, 0)),
            pl.BlockSpec((f, r), lambda i: (0, 0)),
            pl.BlockSpec((f, f), lambda i: (0, 0)),
            pl.BlockSpec((f, 1), lambda i: (0, 0)),
            pl.BlockSpec((f, 1), lambda i: (0, 0)),
            pl.BlockSpec((f, 1), lambda i: (0, 0)),
            pl.BlockSpec((f, 1), lambda i: (0, 0)),
        ]
        out_spec = pl.BlockSpec((f, _BE), lambda i, k=k: (0, i + k * nbk))
        args = (g128, eft, dt, w1ab, w1c, w1d, W2.T, b1c, b2c, gac, bec)
        if k == 0:
            out = pl.pallas_call(
                _mlp_body,
                grid=(nbk,),
                in_specs=base_specs,
                out_specs=out_spec,
                out_shape=jax.ShapeDtypeStruct((f, e), jnp.float32),
                compiler_params=pltpu.CompilerParams(
                    dimension_semantics=("arbitrary",)),
            )(*args)
        else:
            carry_spec = pl.BlockSpec(
                (f, _BE), lambda i, k=k: (0, (k - 1) * nbk))
            out = pl.pallas_call(
                _mlp_body_carry,
                grid=(nbk,),
                in_specs=[carry_spec] + base_specs,
                out_specs=out_spec,
                out_shape=jax.ShapeDtypeStruct((f, e), jnp.float32),
                input_output_aliases={0: 0},
                compiler_params=pltpu.CompilerParams(
                    dimension_semantics=("arbitrary",)),
            )(out, *args)
    return out.T


# speedup vs baseline: 1.4039x; 1.1455x over previous
import dataclasses
import functools

import jax
import jax.numpy as jnp
from jax.experimental import pallas as pl
from jax.experimental.pallas import tpu as pltpu
from jax.experimental.pallas import tpu_sc as plsc

_WINDOW = 128
_WORKERS = 32
_BE = 3200
_K = 2


def _sc_compiler_params():
    cp = pltpu.CompilerParams(use_tc_tiling_on_sc=False)
    if "needs_layout_passes" in pltpu.CompilerParams.__dataclass_fields__:
        cp = dataclasses.replace(cp, needs_layout_passes=False)
    return cp


def _sc_gather(table, eidx, npad, dim):
    mesh = plsc.VectorSubcoreMesh(core_axis_name="core", subcore_axis_name="subcore")
    half = _WINDOW // 2
    nw = npad // _WINDOW // _WORKERS
    assert npad % (_WINDOW * _WORKERS) == 0 and nw % 4 == 0

    @functools.partial(
        pl.kernel,
        out_type=jax.ShapeDtypeStruct((npad, dim), table.dtype),
        mesh=mesh,
        scratch_types=[
            pltpu.VMEM((2, nw * half), jnp.int32),
            pltpu.VMEM((nw * _WINDOW,), jnp.int32),
            pltpu.VMEM((4, _WINDOW, dim), table.dtype),
            pltpu.SemaphoreType.DMA((4,)),
            pltpu.SemaphoreType.DMA((4,)),
        ],
        compiler_params=_sc_compiler_params(),
    )
    def gather_kernel(x_hbm, i_hbm, o_hbm, idx_ref, v_ref, rows_ref, sg, so):
        cid = jax.lax.axis_index("core")
        sid = jax.lax.axis_index("subcore")
        wid = sid * 2 + cid
        base = wid * nw

        pltpu.sync_copy(i_hbm.at[:, pl.ds(base * half, nw * half)], idx_ref)

        @pl.loop(0, nw)
        def _interleave(w):
            for c in range(half // 16):
                pos = w * _WINDOW + c * 32 + jax.lax.iota(jnp.int32, 16) * 2
                plsc.store_scatter(
                    v_ref, [pos], idx_ref.at[0][pl.ds(w * half + c * 16, 16)])
                plsc.store_scatter(
                    v_ref, [pos + 1], idx_ref.at[1][pl.ds(w * half + c * 16, 16)])

        def g_copy(k, b):
            return pltpu.make_async_copy(
                x_hbm.at[v_ref.at[pl.ds(k * _WINDOW, _WINDOW)]],
                rows_ref.at[b], sg.at[b])

        def o_copy(k, b):
            return pltpu.make_async_copy(
                rows_ref.at[b],
                o_hbm.at[pl.ds((base + k) * _WINDOW, _WINDOW)], so.at[b])

        for j in range(3):
            g_copy(j, j).start()

        @pl.loop(0, nw, step=4)
        def _main(g):
            for jj in range(4):
                k = g + jj
                b = jj
                nb = (jj + 3) % 4
                g_copy(k, b).wait()
                o_copy(k, b).start()

                @pl.when(k >= 1)
                def _():
                    o_copy(k - 1, nb).wait()

                @pl.when(k + 3 < nw)
                def _():
                    g_copy(k + 3, nb).start()

        o_copy(nw - 1, (nw - 1) % 4).wait()

    return gather_kernel(table, eidx)


def _mlp_math(g, eft, dt, w1ab, w1c, w1d, w2, b1, b2, gamma, beta):
    x1 = (
        jax.lax.dot_general(w1ab, g, (((1,), (1,)), ((), ())),
                            preferred_element_type=jnp.float32)
        + jnp.dot(w1c, eft, preferred_element_type=jnp.float32)
        + jnp.dot(w1d, dt, preferred_element_type=jnp.float32)
        + b1
    )
    h = x1 * jax.nn.sigmoid(x1)
    x2 = jnp.dot(w2, h, preferred_element_type=jnp.float32) + b2
    x = eft + x2 * jax.nn.sigmoid(x2)
    mu = jnp.mean(x, axis=0, keepdims=True)
    xc = x - mu
    var = jnp.mean(xc * xc, axis=0, keepdims=True)
    y = xc * jax.lax.rsqrt(var + 1e-5)
    return y * gamma + beta


def _mlp_body(g_ref, eft_ref, dt_ref, w1ab_ref, w1c_ref, w1d_ref, w2_ref,
              b1_ref, b2_ref, gamma_ref, beta_ref, out_ref):
    out_ref[...] = _mlp_math(
        g_ref[...], eft_ref[...], dt_ref[...], w1ab_ref[...], w1c_ref[...],
        w1d_ref[...], w2_ref[...], b1_ref[...], b2_ref[...],
        gamma_ref[...], beta_ref[...])


def _mlp_body_carry(carry_ref, g_ref, eft_ref, dt_ref, w1ab_ref, w1c_ref,
                    w1d_ref, w2_ref, b1_ref, b2_ref, gamma_ref, beta_ref,
                    out_ref):
    del carry_ref
    out_ref[...] = _mlp_math(
        g_ref[...], eft_ref[...], dt_ref[...], w1ab_ref[...], w1c_ref[...],
        w1d_ref[...], w2_ref[...], b1_ref[...], b2_ref[...],
        gamma_ref[...], beta_ref[...])


def kernel(node_scalars, edge_feats, d, edge_index, W1, b1, W2, b2, gamma, beta):
    n_nodes, dim = node_scalars.shape
    e, f = edge_feats.shape
    r = d.shape[1]

    eft = edge_feats.T
    dt = d.T
    w1t = W1.T
    w1ab = w1t[:, 0 : 2 * dim]
    w1c = w1t[:, 2 * dim : 2 * dim + f]
    w1d = w1t[:, 2 * dim + f :]
    b1c = b1.reshape(f, 1)
    b2c = b2.reshape(f, 1)
    gac = gamma.reshape(f, 1)
    bec = beta.reshape(f, 1)

    ek = e // _K
    nbk = ek // _BE
    unit = _WINDOW * _WORKERS * 4
    npadk = ((2 * ek + unit - 1) // unit) * unit

    gs = []
    for k in range(_K):
        eidx = edge_index[:, k * ek : (k + 1) * ek]
        eidx = jnp.pad(eidx, ((0, 0), (0, npadk // 2 - ek)))
        gs.append(_sc_gather(node_scalars, eidx, npadk, dim))

    out = None
    for k in range(_K):
        g128 = gs[k].reshape(npadk // 2, 2 * dim)
        base_specs = [
            pl.BlockSpec((_BE, 2 * dim), lambda i: (i, 0)),
            pl.BlockSpec((f, _BE), lambda i, k=k: (0, i + k * nbk)),
            pl.BlockSpec((r, _BE), lambda i, k=k: (0, i + k * nbk)),
            pl.BlockSpec((f, 2 * dim), lambda i: (0, 0)),
            pl.BlockSpec((f, f), lambda i: (0, 0)),
            pl.BlockSpec((f, r), lambda i: (0, 0)),
            pl.BlockSpec((f, f), lambda i: (0, 0)),
            pl.BlockSpec((f, 1), lambda i: (0, 0)),
            pl.BlockSpec((f, 1), lambda i: (0, 0)),
            pl.BlockSpec((f, 1), lambda i: (0, 0)),
            pl.BlockSpec((f, 1), lambda i: (0, 0)),
        ]
        out_spec = pl.BlockSpec((f, _BE), lambda i, k=k: (0, i + k * nbk))
        args = (g128, eft, dt, w1ab, w1c, w1d, W2.T, b1c, b2c, gac, bec)
        if k == 0:
            out = pl.pallas_call(
                _mlp_body,
                grid=(nbk,),
                in_specs=base_specs,
                out_specs=out_spec,
                out_shape=jax.ShapeDtypeStruct((f, e), jnp.float32),
                compiler_params=pltpu.CompilerParams(
                    dimension_semantics=("arbitrary",)),
            )(*args)
        else:
            carry_spec = pl.BlockSpec(
                (f, _BE), lambda i, k=k: (0, (k - 1) * nbk))
            out = pl.pallas_call(
                _mlp_body_carry,
                grid=(nbk,),
                in_specs=[carry_spec] + base_specs,
                out_specs=out_spec,
                out_shape=jax.ShapeDtypeStruct((f, e), jnp.float32),
                input_output_aliases={0: 0},
                compiler_params=pltpu.CompilerParams(
                    dimension_semantics=("arbitrary",)),
            )(out, *args)
    return out.T
